# baseline (speedup 1.0000x reference)
import functools

import jax
import jax.numpy as jnp
from jax import lax
from jax.experimental import pallas as pl
from jax.experimental.pallas import tpu as pltpu
from jax.experimental.pallas import tpu_sc as plsc

N = 1_600_000
NUM_SEG = 50_000
EPS = 0.001

NC = 2
NS = 16
NW = NC * NS
L = 16

SEG_PAD = 51_200
SEG_SLICE = SEG_PAD // NS
N_PAD = 1_638_400
TILE = 2048
ROWS = TILE // 128
K_STEPS = N_PAD // (NW * TILE)
SUB = 800
F32 = jnp.float32
I32 = jnp.int32

_mesh = plsc.VectorSubcoreMesh(core_axis_name="c", subcore_axis_name="s")


def _lane_gather(v, idx):
    dn = lax.GatherDimensionNumbers(offset_dims=(), collapsed_slice_dims=(0,),
                                    start_index_map=(0,))
    return lax.gather(v, idx[:, None], dn, slice_sizes=(1,),
                      mode=lax.GatherScatterMode.PROMISE_IN_BOUNDS)


@functools.partial(
    pl.kernel,
    out_type=jax.ShapeDtypeStruct((NC * SEG_PAD, 4), F32),
    mesh=_mesh,
    scratch_types=(
        [pltpu.VMEM((128, 4), F32) for _ in range(ROWS)]
        + [
            pltpu.VMEM((ROWS, 128), I32),
            pltpu.VMEM_SHARED((SEG_PAD, 4), F32),
            pltpu.SemaphoreType.DMA,
            pltpu.SemaphoreType.DMA,
        ]
    ),
    compiler_params=pltpu.CompilerParams(use_tc_tiling_on_sc=False),
)
def _stats(rows_hbm, b2d_hbm, zero_hbm, part_hbm, *refs):
    rbs = refs[:ROWS]
    idxbuf, acc4, semL, semS = refs[ROWS:]
    c = lax.axis_index("c")
    s = lax.axis_index("s")
    w = c * NS + s

    @pl.when(s == 0)
    def _zero():
        pltpu.sync_copy(zero_hbm, acc4)

    plsc.subcore_barrier()

    def kstep(k, _):
        e0 = (w * K_STEPS + k) * TILE
        r0 = (w * K_STEPS + k) * ROWS
        pltpu.sync_copy(b2d_hbm.at[pl.ds(r0, ROWS)], idxbuf)
        for j in range(ROWS):
            pltpu.async_copy(rows_hbm.at[pl.ds(e0 + j * 128, 128)], rbs[j], semL)
        for j in range(ROWS):
            pltpu.make_async_copy(rows_hbm.at[pl.ds(e0, 128)], rbs[j], semL).wait()
            pltpu.async_copy(rbs[j], acc4.at[idxbuf.at[j]], semS, add=True)
        for j in range(ROWS):
            pltpu.make_async_copy(rbs[j], acc4.at[idxbuf.at[j]], semS).wait()
        return 0

    lax.fori_loop(0, K_STEPS, kstep, 0)
    plsc.subcore_barrier()

    @pl.when(s == 0)
    def _export():
        pltpu.sync_copy(acc4, part_hbm.at[pl.ds(c * SEG_PAD, SEG_PAD)])


def _rsqrt(v):
    bits = lax.bitcast_convert_type(v, I32)
    magic = jnp.full((L,), 0x5F3759DF, I32)
    shift = jnp.full((L,), 1, I32)
    y = lax.bitcast_convert_type(magic - lax.shift_right_logical(bits, shift), F32)
    half = jnp.float32(0.5)
    three_half = jnp.float32(1.5)
    for _ in range(3):
        y = y * (three_half - half * v * y * y)
    return y


@functools.partial(
    pl.kernel,
    out_type=jax.ShapeDtypeStruct((N_PAD,), F32),
    mesh=_mesh,
    scratch_types=[
        pltpu.VMEM((SEG_SLICE,), I32),
        pltpu.VMEM((SUB * 4,), F32),
        pltpu.VMEM((SUB * 4,), F32),
        pltpu.VMEM((TILE,), F32),
        pltpu.VMEM((TILE,), I32),
        pltpu.VMEM((TILE,), I32),
        pltpu.VMEM((TILE,), F32),
        pltpu.VMEM((L,), F32),
        pltpu.VMEM((L,), F32),
        pltpu.VMEM_SHARED((SEG_PAD,), I32),
        pltpu.SemaphoreType.DMA,
    ],
)
def _norm(x_hbm, b_hbm, part_hbm, gain_hbm, bias_hbm, out_hbm,
          ab_v, q0, q1, xbuf, idxbuf, pbuf, obuf,
          gb, bb, ab_sp, sem):
    c = lax.axis_index("c")
    s = lax.axis_index("s")
    w = c * NS + s

    pltpu.sync_copy(gain_hbm, gb)
    pltpu.sync_copy(bias_hbm, bb)
    gain = gb[...]
    bias = bb[...]

    lane = lax.iota(I32, L)
    fifteen = jnp.full((L,), 15, I32)
    m4 = lane < 4
    m8 = lane < 8
    m12 = lane < 12
    perm = {}
    for o in range(3):
        for kk in range(4):
            perm[(o, kk)] = (lane * 4 + jnp.full((L,), o - 16 * kk, I32)) & fifteen

    def _deint(qref, i, o):
        g = []
        for kk in range(4):
            v = qref[pl.ds(i * 64 + 16 * kk, L)]
            g.append(_lane_gather(v, perm[(o, kk)]))
        return jnp.where(m4, g[0], jnp.where(m8, g[1], jnp.where(m12, g[2], g[3])))

    for sub in range(SEG_SLICE // SUB):
        sb = s * SEG_SLICE + sub * SUB
        pltpu.sync_copy(part_hbm.at[pl.ds(sb * 4, SUB * 4)], q0)
        pltpu.sync_copy(part_hbm.at[pl.ds((SEG_PAD + sb) * 4, SUB * 4)], q1)

        def seg(i, _):
            sm = _deint(q0, i, 0) + _deint(q1, i, 0)
            q = _deint(q0, i, 1) + _deint(q1, i, 1)
            cnt = _deint(q0, i, 2) + _deint(q1, i, 2)
            cl = jnp.maximum(cnt, jnp.float32(1.0))
            m = sm / cl
            var = jnp.maximum(q / cl - m * m, jnp.float32(0.0))
            vs = jnp.maximum(var, jnp.float32(1e-30))
            std = vs * _rsqrt(vs)
            a = gain / (std + jnp.float32(EPS))
            b = bias - m * a
            half = jnp.full((L,), 0x8000, I32)
            himask = jnp.full((L,), -65536, I32)
            s16 = jnp.full((L,), 16, I32)
            ar = lax.bitcast_convert_type(a, I32) + half
            br = lax.bitcast_convert_type(b, I32) + half
            packed = (ar & himask) | lax.shift_right_logical(br, s16)
            ab_v[pl.ds(sub * SUB + i * L, L)] = packed
            return 0

        lax.fori_loop(0, SUB // L, seg, 0)

    sl = pl.ds(s * SEG_SLICE, SEG_SLICE)
    pltpu.sync_copy(ab_v, ab_sp.at[sl])
    plsc.subcore_barrier()

    def kstep(k, _):
        e0 = (w * K_STEPS + k) * TILE
        pltpu.sync_copy(x_hbm.at[pl.ds(e0, TILE)], xbuf)
        pltpu.sync_copy(b_hbm.at[pl.ds(e0, TILE)], idxbuf)

        def fire(j, _):
            d = pl.ds(j * 128, 128)
            idxr = idxbuf.at[d]
            pltpu.async_copy(ab_sp.at[idxr], pbuf.at[d], sem)
            return 0

        lax.fori_loop(0, ROWS, fire, 0)

        def drain(j, _):
            d = pl.ds(j * 128, 128)
            idxr = idxbuf.at[d]
            pltpu.make_async_copy(ab_sp.at[idxr], pbuf.at[d], sem).wait()
            return 0

        lax.fori_loop(0, ROWS, drain, 0)

        himask = jnp.full((L,), -65536, I32)
        s16 = jnp.full((L,), 16, I32)

        def inner(i, _):
            d = pl.ds(i * L, L)
            p = pbuf[d]
            a = lax.bitcast_convert_type(p & himask, F32)
            b = lax.bitcast_convert_type(lax.shift_left(p, s16), F32)
            obuf[d] = xbuf[d] * a + b
            return 0

        lax.fori_loop(0, TILE // L, inner, 0)
        pltpu.sync_copy(obuf, out_hbm.at[pl.ds(e0, TILE)])
        return 0

    lax.fori_loop(0, K_STEPS, kstep, 0)


def kernel(inputs, batch, gain, bias):
    x = inputs.astype(F32)
    b = batch.astype(I32)
    pad = N_PAD - N
    xp = jnp.concatenate([x, jnp.zeros((pad,), F32)])
    dummy = NUM_SEG + (jnp.arange(pad, dtype=I32) % (SEG_PAD - NUM_SEG))
    bp = jnp.concatenate([b, dummy])
    b2d = bp.reshape(N_PAD // 128, 128)
    rows4 = jnp.stack(
        [xp, xp * xp, jnp.ones((N_PAD,), F32), jnp.zeros((N_PAD,), F32)], axis=-1)
    zeros4 = jnp.zeros((SEG_PAD, 4), F32)
    g16 = jnp.broadcast_to(gain.astype(F32), (L,))
    b16 = jnp.broadcast_to(bias.astype(F32), (L,))
    part = _stats(rows4, b2d, zeros4)
    outp = _norm(xp, bp, part.reshape(-1), g16, b16)
    return outp[:N].reshape(N, 1)


# speedup vs baseline: 12.5290x; 12.5290x over previous
import functools

import jax
import jax.numpy as jnp
from jax import lax
from jax.experimental import pallas as pl
from jax.experimental.pallas import tpu as pltpu
from jax.experimental.pallas import tpu_sc as plsc

N = 1_600_000
NUM_SEG = 50_000
EPS = 0.001

NC = 2
NS = 16
NW = NC * NS
L = 16

SEG_PAD = 51_200
SEG_SLICE = SEG_PAD // NS
N_PAD = 1_638_400
TILE = 2048
ROWS = TILE // 128
K_STEPS = N_PAD // (NW * TILE)
SUB = 800
NSET = 4
F32 = jnp.float32
I32 = jnp.int32

_mesh = plsc.VectorSubcoreMesh(core_axis_name="c", subcore_axis_name="s")


def _rsqrt(v):
    bits = lax.bitcast_convert_type(v, I32)
    magic = jnp.full((L,), 0x5F3759DF, I32)
    shift = jnp.full((L,), 1, I32)
    y = lax.bitcast_convert_type(magic - lax.shift_right_logical(bits, shift), F32)
    half = jnp.float32(0.5)
    three_half = jnp.float32(1.5)
    for _ in range(3):
        y = y * (three_half - half * v * y * y)
    return y


@functools.partial(
    pl.kernel,
    out_type=jax.ShapeDtypeStruct((NC * 3 * SEG_PAD,), F32),
    mesh=_mesh,
    scratch_types=(
        [pltpu.VMEM((TILE,), F32) for _ in range(NSET)]
        + [pltpu.VMEM((TILE,), F32) for _ in range(NSET)]
        + [pltpu.VMEM((ROWS, 128), I32) for _ in range(NSET)]
        + [
            pltpu.VMEM((128,), F32),
            pltpu.VMEM((SEG_SLICE,), F32),
            pltpu.VMEM_SHARED((SEG_PAD,), F32),
            pltpu.VMEM_SHARED((SEG_PAD,), F32),
            pltpu.VMEM_SHARED((SEG_PAD,), F32),
            pltpu.SemaphoreType.DMA,
            pltpu.SemaphoreType.DMA,
        ]
    ),
)
def _stats(x_hbm, b2d_hbm, part_hbm, *refs):
    xbufs = refs[0:NSET]
    xsqs = refs[NSET:2 * NSET]
    idxbufs = refs[2 * NSET:3 * NSET]
    ones, zbuf, acc_c, acc_s, acc_q, semL, semS = refs[3 * NSET:]
    c = lax.axis_index("c")
    s = lax.axis_index("s")
    w = c * NS + s
    base = s * SEG_SLICE

    onev = jnp.full((L,), 1.0, F32)
    zerov = jnp.zeros((L,), F32)

    def fill_ones(i, _):
        ones[pl.ds(i * L, L)] = onev
        return 0

    lax.fori_loop(0, 128 // L, fill_ones, 0)

    def fill_z(i, _):
        zbuf[pl.ds(i * L, L)] = zerov
        return 0

    lax.fori_loop(0, SEG_SLICE // L, fill_z, 0)
    pltpu.sync_copy(zbuf, acc_c.at[pl.ds(base, SEG_SLICE)])
    pltpu.sync_copy(zbuf, acc_s.at[pl.ds(base, SEG_SLICE)])
    pltpu.sync_copy(zbuf, acc_q.at[pl.ds(base, SEG_SLICE)])
    plsc.subcore_barrier()

    def fire_loads(k):
        e0 = (w * K_STEPS + k) * TILE
        r0 = (w * K_STEPS + k) * ROWS
        st = k % NSET
        pltpu.async_copy(x_hbm.at[pl.ds(e0, TILE)], xbufs[st], semL)
        pltpu.async_copy(b2d_hbm.at[pl.ds(r0, ROWS)], idxbufs[st], semL)

    def wait_loads(k):
        e0 = (w * K_STEPS + k) * TILE
        r0 = (w * K_STEPS + k) * ROWS
        st = k % NSET
        pltpu.make_async_copy(x_hbm.at[pl.ds(e0, TILE)], xbufs[st], semL).wait()
        pltpu.make_async_copy(b2d_hbm.at[pl.ds(r0, ROWS)], idxbufs[st], semL).wait()

    def fire_scatters(k):
        st = k % NSET

        def fire(j, _):
            idxr = idxbufs[st].at[j]
            pltpu.async_copy(ones, acc_c.at[idxr], semS, add=True)
            pltpu.async_copy(xbufs[st].at[pl.ds(j * 128, 128)], acc_s.at[idxr],
                             semS, add=True)
            pltpu.async_copy(xsqs[st].at[pl.ds(j * 128, 128)], acc_q.at[idxr],
                             semS, add=True)
            return 0

        lax.fori_loop(0, ROWS, fire, 0)

    def drain_scatters(k):
        st = k % NSET

        def drain(j, _):
            idxr = idxbufs[st].at[j]
            pltpu.make_async_copy(ones, acc_c.at[idxr], semS).wait()
            pltpu.make_async_copy(xbufs[st].at[pl.ds(j * 128, 128)],
                                  acc_s.at[idxr], semS).wait()
            pltpu.make_async_copy(xsqs[st].at[pl.ds(j * 128, 128)],
                                  acc_q.at[idxr], semS).wait()
            return 0

        lax.fori_loop(0, ROWS, drain, 0)

    def square(k):
        st = k % NSET

        def sq(i, _):
            v = xbufs[st][pl.ds(i * L, L)]
            xsqs[st][pl.ds(i * L, L)] = v * v
            return 0

        lax.fori_loop(0, TILE // L, sq, 0)

    fire_loads(0)
    fire_loads(1)
    for k in range(K_STEPS):
        wait_loads(k)
        square(k)
        if k >= 2:
            drain_scatters(k - 2)
        if k + 2 < K_STEPS:
            fire_loads(k + 2)
        fire_scatters(k)
    drain_scatters(K_STEPS - 2)
    drain_scatters(K_STEPS - 1)
    plsc.subcore_barrier()

    pltpu.sync_copy(acc_c.at[pl.ds(base, SEG_SLICE)], zbuf)
    pltpu.sync_copy(zbuf, part_hbm.at[pl.ds(c * 3 * SEG_PAD + base, SEG_SLICE)])
    pltpu.sync_copy(acc_s.at[pl.ds(base, SEG_SLICE)], zbuf)
    pltpu.sync_copy(zbuf, part_hbm.at[pl.ds((c * 3 + 1) * SEG_PAD + base, SEG_SLICE)])
    pltpu.sync_copy(acc_q.at[pl.ds(base, SEG_SLICE)], zbuf)
    pltpu.sync_copy(zbuf, part_hbm.at[pl.ds((c * 3 + 2) * SEG_PAD + base, SEG_SLICE)])


@functools.partial(
    pl.kernel,
    out_type=jax.ShapeDtypeStruct((N_PAD,), F32),
    mesh=_mesh,
    scratch_types=(
        [pltpu.VMEM((TILE,), F32) for _ in range(2)]
        + [pltpu.VMEM((TILE,), I32) for _ in range(2)]
        + [pltpu.VMEM((TILE,), I32) for _ in range(2)]
        + [pltpu.VMEM((TILE,), F32) for _ in range(2)]
        + [
            pltpu.VMEM((SEG_SLICE,), I32),
            pltpu.VMEM((SUB,), F32),
            pltpu.VMEM((SUB,), F32),
            pltpu.VMEM((SUB,), F32),
            pltpu.VMEM((SUB,), F32),
            pltpu.VMEM((SUB,), F32),
            pltpu.VMEM((SUB,), F32),
            pltpu.VMEM((L,), F32),
            pltpu.VMEM((L,), F32),
            pltpu.VMEM_SHARED((SEG_PAD,), I32),
            pltpu.SemaphoreType.DMA,
            pltpu.SemaphoreType.DMA,
            pltpu.SemaphoreType.DMA,
        ]
    ),
)
def _norm(x_hbm, b_hbm, part_hbm, gain_hbm, bias_hbm, out_hbm, *refs):
    xbufs = refs[0:2]
    idxbufs = refs[2:4]
    pbufs = refs[4:6]
    obufs = refs[6:8]
    (ab_v, p0, p1, p2, p3, p4, p5, gb, bb, ab_sp, semL, semG, semO) = refs[8:]
    c = lax.axis_index("c")
    s = lax.axis_index("s")
    w = c * NS + s

    pltpu.sync_copy(gain_hbm, gb)
    pltpu.sync_copy(bias_hbm, bb)
    gain = gb[...]
    bias = bb[...]

    for sub in range(SEG_SLICE // SUB):
        sb = s * SEG_SLICE + sub * SUB
        pltpu.sync_copy(part_hbm.at[pl.ds(0 * SEG_PAD + sb, SUB)], p0)
        pltpu.sync_copy(part_hbm.at[pl.ds(3 * SEG_PAD + sb, SUB)], p1)
        pltpu.sync_copy(part_hbm.at[pl.ds(1 * SEG_PAD + sb, SUB)], p2)
        pltpu.sync_copy(part_hbm.at[pl.ds(4 * SEG_PAD + sb, SUB)], p3)
        pltpu.sync_copy(part_hbm.at[pl.ds(2 * SEG_PAD + sb, SUB)], p4)
        pltpu.sync_copy(part_hbm.at[pl.ds(5 * SEG_PAD + sb, SUB)], p5)

        def seg(i, _):
            d = pl.ds(i * L, L)
            cnt = p0[d] + p1[d]
            cl = jnp.maximum(cnt, jnp.float32(1.0))
            sm = p2[d] + p3[d]
            q = p4[d] + p5[d]
            m = sm / cl
            var = jnp.maximum(q / cl - m * m, jnp.float32(0.0))
            vs = jnp.maximum(var, jnp.float32(1e-30))
            std = vs * _rsqrt(vs)
            a = gain / (std + jnp.float32(EPS))
            b = bias - m * a
            half = jnp.full((L,), 0x8000, I32)
            himask = jnp.full((L,), -65536, I32)
            s16 = jnp.full((L,), 16, I32)
            ar = lax.bitcast_convert_type(a, I32) + half
            br = lax.bitcast_convert_type(b, I32) + half
            packed = (ar & himask) | lax.shift_right_logical(br, s16)
            ab_v[pl.ds(sub * SUB + i * L, L)] = packed
            return 0

        lax.fori_loop(0, SUB // L, seg, 0)

    sl = pl.ds(s * SEG_SLICE, SEG_SLICE)
    pltpu.sync_copy(ab_v, ab_sp.at[sl])
    plsc.subcore_barrier()

    himask = jnp.full((L,), -65536, I32)
    s16 = jnp.full((L,), 16, I32)

    def fire_loads(k):
        e0 = (w * K_STEPS + k) * TILE
        st = k % 2
        pltpu.async_copy(x_hbm.at[pl.ds(e0, TILE)], xbufs[st], semL)
        pltpu.async_copy(b_hbm.at[pl.ds(e0, TILE)], idxbufs[st], semL)

    def wait_loads(k):
        e0 = (w * K_STEPS + k) * TILE
        st = k % 2
        pltpu.make_async_copy(x_hbm.at[pl.ds(e0, TILE)], xbufs[st], semL).wait()
        pltpu.make_async_copy(b_hbm.at[pl.ds(e0, TILE)], idxbufs[st], semL).wait()

    def fire_gathers(k):
        st = k % 2

        def fire(j, _):
            d = pl.ds(j * 128, 128)
            pltpu.async_copy(ab_sp.at[idxbufs[st].at[d]], pbufs[st].at[d], semG)
            return 0

        lax.fori_loop(0, ROWS, fire, 0)

    def drain_gathers(k):
        st = k % 2

        def drain(j, _):
            d = pl.ds(j * 128, 128)
            pltpu.make_async_copy(ab_sp.at[idxbufs[st].at[d]], pbufs[st].at[d],
                                  semG).wait()
            return 0

        lax.fori_loop(0, ROWS, drain, 0)

    def compute_store(k):
        st = k % 2
        e0 = (w * K_STEPS + k) * TILE

        def inner(i, _):
            d = pl.ds(i * L, L)
            p = pbufs[st][d]
            a = lax.bitcast_convert_type(p & himask, F32)
            b = lax.bitcast_convert_type(lax.shift_left(p, s16), F32)
            obufs[st][d] = xbufs[st][d] * a + b
            return 0

        lax.fori_loop(0, TILE // L, inner, 0)
        pltpu.async_copy(obufs[st], out_hbm.at[pl.ds(e0, TILE)], semO)

    def wait_store(k):
        st = k % 2
        e0 = (w * K_STEPS + k) * TILE
        pltpu.make_async_copy(obufs[st], out_hbm.at[pl.ds(e0, TILE)], semO).wait()

    fire_loads(0)
    for k in range(K_STEPS):
        wait_loads(k)
        fire_gathers(k)
        if k + 1 < K_STEPS:
            fire_loads(k + 1)
        drain_gathers(k)
        if k >= 2:
            wait_store(k - 2)
        compute_store(k)
    wait_store(K_STEPS - 2)
    wait_store(K_STEPS - 1)


def kernel(inputs, batch, gain, bias):
    x = inputs.astype(F32)
    b = batch.astype(I32)
    pad = N_PAD - N
    xp = jnp.concatenate([x, jnp.zeros((pad,), F32)])
    dummy = NUM_SEG + (jnp.arange(pad, dtype=I32) % (SEG_PAD - NUM_SEG))
    bp = jnp.concatenate([b, dummy])
    b2d = bp.reshape(N_PAD // 128, 128)
    g16 = jnp.broadcast_to(gain.astype(F32), (L,))
    b16 = jnp.broadcast_to(bias.astype(F32), (L,))
    part = _stats(xp, b2d)
    outp = _norm(xp, bp, part, g16, b16)
    return outp[:N].reshape(N, 1)


# speedup vs baseline: 13.0730x; 1.0434x over previous
import functools

import jax
import jax.numpy as jnp
from jax import lax
from jax.experimental import pallas as pl
from jax.experimental.pallas import tpu as pltpu
from jax.experimental.pallas import tpu_sc as plsc

N = 1_600_000
NUM_SEG = 50_000
EPS = 0.001

NC = 2
NS = 16
NW = NC * NS
L = 16

SEG_PAD = 51_200
SEG_SLICE = SEG_PAD // NS
N_PAD = 1_638_400
TILE = 2048
ROWS = TILE // 128
K_STEPS = N_PAD // (NW * TILE)
SUB = 800
NSET = 4
F32 = jnp.float32
I32 = jnp.int32

_mesh = plsc.VectorSubcoreMesh(core_axis_name="c", subcore_axis_name="s")


def _rsqrt(v):
    bits = lax.bitcast_convert_type(v, I32)
    magic = jnp.full((L,), 0x5F3759DF, I32)
    shift = jnp.full((L,), 1, I32)
    y = lax.bitcast_convert_type(magic - lax.shift_right_logical(bits, shift), F32)
    half = jnp.float32(0.5)
    three_half = jnp.float32(1.5)
    for _ in range(3):
        y = y * (three_half - half * v * y * y)
    return y


@functools.partial(
    pl.kernel,
    out_type=jax.ShapeDtypeStruct((NC * 3 * SEG_PAD,), F32),
    mesh=_mesh,
    scratch_types=(
        [pltpu.VMEM((TILE,), F32) for _ in range(NSET)]
        + [pltpu.VMEM((TILE,), F32) for _ in range(NSET)]
        + [pltpu.VMEM((ROWS, 128), I32) for _ in range(NSET)]
        + [
            pltpu.VMEM((128,), F32),
            pltpu.VMEM((SEG_SLICE,), F32),
            pltpu.VMEM_SHARED((SEG_PAD,), F32),
            pltpu.VMEM_SHARED((SEG_PAD,), F32),
            pltpu.VMEM_SHARED((SEG_PAD,), F32),
            pltpu.SemaphoreType.DMA,
            pltpu.SemaphoreType.DMA,
        ]
    ),
)
def _stats(x_hbm, b2d_hbm, part_hbm, *refs):
    xbufs = refs[0:NSET]
    xsqs = refs[NSET:2 * NSET]
    idxbufs = refs[2 * NSET:3 * NSET]
    ones, zbuf, acc_c, acc_s, acc_q, semL, semS = refs[3 * NSET:]
    c = lax.axis_index("c")
    s = lax.axis_index("s")
    w = c * NS + s
    base = s * SEG_SLICE

    onev = jnp.full((L,), 1.0, F32)
    zerov = jnp.zeros((L,), F32)

    def fill_ones(i, _):
        ones[pl.ds(i * L, L)] = onev
        return 0

    lax.fori_loop(0, 128 // L, fill_ones, 0)

    def fill_z(i, _):
        zbuf[pl.ds(i * L, L)] = zerov
        return 0

    lax.fori_loop(0, SEG_SLICE // L, fill_z, 0)
    pltpu.sync_copy(zbuf, acc_c.at[pl.ds(base, SEG_SLICE)])
    pltpu.sync_copy(zbuf, acc_s.at[pl.ds(base, SEG_SLICE)])
    pltpu.sync_copy(zbuf, acc_q.at[pl.ds(base, SEG_SLICE)])
    plsc.subcore_barrier()

    def fire_loads(k):
        e0 = (w * K_STEPS + k) * TILE
        r0 = (w * K_STEPS + k) * ROWS
        st = k % NSET
        pltpu.async_copy(x_hbm.at[pl.ds(e0, TILE)], xbufs[st], semL)
        pltpu.async_copy(b2d_hbm.at[pl.ds(r0, ROWS)], idxbufs[st], semL)

    def wait_loads(k):
        e0 = (w * K_STEPS + k) * TILE
        r0 = (w * K_STEPS + k) * ROWS
        st = k % NSET
        pltpu.make_async_copy(x_hbm.at[pl.ds(e0, TILE)], xbufs[st], semL).wait()
        pltpu.make_async_copy(b2d_hbm.at[pl.ds(r0, ROWS)], idxbufs[st], semL).wait()

    def fire_scatters(k):
        st = k % NSET

        def fire(j, _):
            idxr = idxbufs[st].at[j]
            pltpu.async_copy(ones, acc_c.at[idxr], semS, add=True)
            pltpu.async_copy(xbufs[st].at[pl.ds(j * 128, 128)], acc_s.at[idxr],
                             semS, add=True)
            pltpu.async_copy(xsqs[st].at[pl.ds(j * 128, 128)], acc_q.at[idxr],
                             semS, add=True)
            return 0

        lax.fori_loop(0, ROWS, fire, 0)

    def drain_scatters(k):
        st = k % NSET

        def drain(j, _):
            idxr = idxbufs[st].at[j]
            pltpu.make_async_copy(ones, acc_c.at[idxr], semS).wait()
            pltpu.make_async_copy(xbufs[st].at[pl.ds(j * 128, 128)],
                                  acc_s.at[idxr], semS).wait()
            pltpu.make_async_copy(xsqs[st].at[pl.ds(j * 128, 128)],
                                  acc_q.at[idxr], semS).wait()
            return 0

        lax.fori_loop(0, ROWS, drain, 0)

    def square(k):
        st = k % NSET

        def sq(i, _):
            v = xbufs[st][pl.ds(i * L, L)]
            xsqs[st][pl.ds(i * L, L)] = v * v
            return 0

        lax.fori_loop(0, TILE // L, sq, 0)

    fire_loads(0)
    fire_loads(1)
    for k in range(K_STEPS):
        wait_loads(k)
        square(k)
        if k >= 2:
            drain_scatters(k - 2)
        if k + 2 < K_STEPS:
            fire_loads(k + 2)
        fire_scatters(k)
    drain_scatters(K_STEPS - 2)
    drain_scatters(K_STEPS - 1)
    plsc.subcore_barrier()

    pltpu.sync_copy(acc_c.at[pl.ds(base, SEG_SLICE)], zbuf)
    pltpu.sync_copy(zbuf, part_hbm.at[pl.ds(c * 3 * SEG_PAD + base, SEG_SLICE)])
    pltpu.sync_copy(acc_s.at[pl.ds(base, SEG_SLICE)], zbuf)
    pltpu.sync_copy(zbuf, part_hbm.at[pl.ds((c * 3 + 1) * SEG_PAD + base, SEG_SLICE)])
    pltpu.sync_copy(acc_q.at[pl.ds(base, SEG_SLICE)], zbuf)
    pltpu.sync_copy(zbuf, part_hbm.at[pl.ds((c * 3 + 2) * SEG_PAD + base, SEG_SLICE)])


@functools.partial(
    pl.kernel,
    out_type=jax.ShapeDtypeStruct((N,), F32),
    mesh=_mesh,
    scratch_types=(
        [pltpu.VMEM((TILE,), F32) for _ in range(2)]
        + [pltpu.VMEM((TILE,), I32) for _ in range(2)]
        + [pltpu.VMEM((TILE,), I32) for _ in range(2)]
        + [pltpu.VMEM((TILE,), F32) for _ in range(2)]
        + [
            pltpu.VMEM((SEG_SLICE,), I32),
            pltpu.VMEM((SUB,), F32),
            pltpu.VMEM((SUB,), F32),
            pltpu.VMEM((SUB,), F32),
            pltpu.VMEM((SUB,), F32),
            pltpu.VMEM((SUB,), F32),
            pltpu.VMEM((SUB,), F32),
            pltpu.VMEM((L,), F32),
            pltpu.VMEM((L,), F32),
            pltpu.VMEM_SHARED((SEG_PAD,), I32),
            pltpu.SemaphoreType.DMA,
            pltpu.SemaphoreType.DMA,
            pltpu.SemaphoreType.DMA,
        ]
    ),
)
def _norm(x_hbm, b_hbm, part_hbm, gain_hbm, bias_hbm, out_hbm, *refs):
    xbufs = refs[0:2]
    idxbufs = refs[2:4]
    pbufs = refs[4:6]
    obufs = refs[6:8]
    (ab_v, p0, p1, p2, p3, p4, p5, gb, bb, ab_sp, semL, semG, semO) = refs[8:]
    c = lax.axis_index("c")
    s = lax.axis_index("s")
    w = c * NS + s

    pltpu.sync_copy(gain_hbm, gb)
    pltpu.sync_copy(bias_hbm, bb)
    gain = gb[...]
    bias = bb[...]

    for sub in range(SEG_SLICE // SUB):
        sb = s * SEG_SLICE + sub * SUB
        pltpu.sync_copy(part_hbm.at[pl.ds(0 * SEG_PAD + sb, SUB)], p0)
        pltpu.sync_copy(part_hbm.at[pl.ds(3 * SEG_PAD + sb, SUB)], p1)
        pltpu.sync_copy(part_hbm.at[pl.ds(1 * SEG_PAD + sb, SUB)], p2)
        pltpu.sync_copy(part_hbm.at[pl.ds(4 * SEG_PAD + sb, SUB)], p3)
        pltpu.sync_copy(part_hbm.at[pl.ds(2 * SEG_PAD + sb, SUB)], p4)
        pltpu.sync_copy(part_hbm.at[pl.ds(5 * SEG_PAD + sb, SUB)], p5)

        def seg(i, _):
            d = pl.ds(i * L, L)
            cnt = p0[d] + p1[d]
            cl = jnp.maximum(cnt, jnp.float32(1.0))
            sm = p2[d] + p3[d]
            q = p4[d] + p5[d]
            m = sm / cl
            var = jnp.maximum(q / cl - m * m, jnp.float32(0.0))
            vs = jnp.maximum(var, jnp.float32(1e-30))
            std = vs * _rsqrt(vs)
            a = gain / (std + jnp.float32(EPS))
            b = bias - m * a
            half = jnp.full((L,), 0x8000, I32)
            himask = jnp.full((L,), -65536, I32)
            s16 = jnp.full((L,), 16, I32)
            ar = lax.bitcast_convert_type(a, I32) + half
            br = lax.bitcast_convert_type(b, I32) + half
            packed = (ar & himask) | lax.shift_right_logical(br, s16)
            ab_v[pl.ds(sub * SUB + i * L, L)] = packed
            return 0

        lax.fori_loop(0, SUB // L, seg, 0)

    sl = pl.ds(s * SEG_SLICE, SEG_SLICE)
    pltpu.sync_copy(ab_v, ab_sp.at[sl])
    plsc.subcore_barrier()

    himask = jnp.full((L,), -65536, I32)
    s16 = jnp.full((L,), 16, I32)

    def fire_loads(k):
        e0 = (w * K_STEPS + k) * TILE
        st = k % 2
        pltpu.async_copy(x_hbm.at[pl.ds(e0, TILE)], xbufs[st], semL)
        pltpu.async_copy(b_hbm.at[pl.ds(e0, TILE)], idxbufs[st], semL)

    def wait_loads(k):
        e0 = (w * K_STEPS + k) * TILE
        st = k % 2
        pltpu.make_async_copy(x_hbm.at[pl.ds(e0, TILE)], xbufs[st], semL).wait()
        pltpu.make_async_copy(b_hbm.at[pl.ds(e0, TILE)], idxbufs[st], semL).wait()

    def fire_gathers(k, lo, hi):
        st = k % 2

        def fire(j, _):
            d = pl.ds(j * 128, 128)
            pltpu.async_copy(ab_sp.at[idxbufs[st].at[d]], pbufs[st].at[d], semG)
            return 0

        lax.fori_loop(lo, hi, fire, 0)

    def drain_gathers(k, lo, hi):
        st = k % 2

        def drain(j, _):
            d = pl.ds(j * 128, 128)
            pltpu.make_async_copy(ab_sp.at[idxbufs[st].at[d]], pbufs[st].at[d],
                                  semG).wait()
            return 0

        lax.fori_loop(lo, hi, drain, 0)

    def compute(k, lo, hi):
        st = k % 2

        def inner(i, _):
            d = pl.ds(i * L, L)
            p = pbufs[st][d]
            a = lax.bitcast_convert_type(p & himask, F32)
            b = lax.bitcast_convert_type(lax.shift_left(p, s16), F32)
            obufs[st][d] = xbufs[st][d] * a + b
            return 0

        lax.fori_loop(lo, hi, inner, 0)

    STRAD = 512

    def fire_store(k):
        st = k % 2
        e0 = (w * K_STEPS + k) * TILE

        @pl.when(e0 + TILE <= N)
        def _full():
            pltpu.async_copy(obufs[st], out_hbm.at[pl.ds(e0, TILE)], semO)

        @pl.when(jnp.logical_and(e0 < N, e0 + TILE > N))
        def _part():
            pltpu.async_copy(obufs[st].at[pl.ds(0, STRAD)],
                             out_hbm.at[pl.ds(e0, STRAD)], semO)

    def wait_store(k):
        st = k % 2
        e0 = (w * K_STEPS + k) * TILE

        @pl.when(e0 + TILE <= N)
        def _full():
            pltpu.make_async_copy(obufs[st], out_hbm.at[pl.ds(e0, TILE)],
                                  semO).wait()

        @pl.when(jnp.logical_and(e0 < N, e0 + TILE > N))
        def _part():
            pltpu.make_async_copy(obufs[st].at[pl.ds(0, STRAD)],
                                  out_hbm.at[pl.ds(e0, STRAD)], semO).wait()

    HALF = ROWS // 2
    fire_loads(0)
    for k in range(K_STEPS):
        wait_loads(k)
        fire_gathers(k, 0, HALF)
        if k + 1 < K_STEPS:
            fire_loads(k + 1)
        drain_gathers(k, 0, HALF)
        fire_gathers(k, HALF, ROWS)
        compute(k, 0, TILE // (2 * L))
        drain_gathers(k, HALF, ROWS)
        if k >= 2:
            wait_store(k - 2)
        compute(k, TILE // (2 * L), TILE // L)
        fire_store(k)
    wait_store(K_STEPS - 2)
    wait_store(K_STEPS - 1)


def kernel(inputs, batch, gain, bias):
    x = inputs.astype(F32)
    b = batch.astype(I32)
    pad = N_PAD - N
    xp = jnp.concatenate([x, jnp.zeros((pad,), F32)])
    dummy = NUM_SEG + (jnp.arange(pad, dtype=I32) % (SEG_PAD - NUM_SEG))
    bp = jnp.concatenate([b, dummy])
    b2d = bp.reshape(N_PAD // 128, 128)
    g16 = jnp.broadcast_to(gain.astype(F32), (L,))
    b16 = jnp.broadcast_to(bias.astype(F32), (L,))
    part = _stats(xp, b2d)
    outp = _norm(xp, bp, part, g16, b16)
    return outp.reshape(N, 1)


# speedup vs baseline: 13.7204x; 1.0495x over previous
import functools

import jax
import jax.numpy as jnp
from jax import lax
from jax.experimental import pallas as pl
from jax.experimental.pallas import tpu as pltpu
from jax.experimental.pallas import tpu_sc as plsc

N = 1_600_000
NUM_SEG = 50_000
EPS = 0.001

NC = 2
NS = 16
NW = NC * NS
L = 16

SEG_PAD = 51_200
SEG_SLICE = SEG_PAD // NS
N_PAD = 1_638_400
TILE = 2048
ROWS = TILE // 128
K_STEPS = N_PAD // (NW * TILE)
SUB = 800
NSET = 4
F32 = jnp.float32
I32 = jnp.int32

_mesh = plsc.VectorSubcoreMesh(core_axis_name="c", subcore_axis_name="s")


def _rsqrt(v):
    bits = lax.bitcast_convert_type(v, I32)
    magic = jnp.full((L,), 0x5F3759DF, I32)
    shift = jnp.full((L,), 1, I32)
    y = lax.bitcast_convert_type(magic - lax.shift_right_logical(bits, shift), F32)
    half = jnp.float32(0.5)
    three_half = jnp.float32(1.5)
    for _ in range(3):
        y = y * (three_half - half * v * y * y)
    return y


@functools.partial(
    pl.kernel,
    out_type=jax.ShapeDtypeStruct((NC * 3 * SEG_PAD,), F32),
    mesh=_mesh,
    scratch_types=(
        [pltpu.VMEM((TILE,), F32) for _ in range(NSET)]
        + [pltpu.VMEM((TILE,), F32) for _ in range(NSET)]
        + [pltpu.VMEM((ROWS, 128), I32) for _ in range(NSET)]
        + [
            pltpu.VMEM((128,), F32),
            pltpu.VMEM((SEG_SLICE,), F32),
            pltpu.VMEM_SHARED((SEG_PAD,), F32),
            pltpu.VMEM_SHARED((SEG_PAD,), F32),
            pltpu.VMEM_SHARED((SEG_PAD,), F32),
            pltpu.SemaphoreType.DMA,
            pltpu.SemaphoreType.DMA,
        ]
    ),
)
def _stats(x_hbm, b2d_hbm, part_hbm, *refs):
    xbufs = refs[0:NSET]
    xsqs = refs[NSET:2 * NSET]
    idxbufs = refs[2 * NSET:3 * NSET]
    ones, zbuf, acc_c, acc_s, acc_q, semL, semS = refs[3 * NSET:]
    c = lax.axis_index("c")
    s = lax.axis_index("s")
    w = c * NS + s
    base = s * SEG_SLICE

    def fire_loads(k):
        e0 = (w * K_STEPS + k) * TILE
        ex = jnp.minimum(e0, N - TILE)
        r0 = (w * K_STEPS + k) * ROWS
        st = k % NSET
        pltpu.async_copy(x_hbm.at[pl.ds(ex, TILE)], xbufs[st], semL)
        pltpu.async_copy(b2d_hbm.at[pl.ds(r0, ROWS)], idxbufs[st], semL)

    def wait_loads(k):
        e0 = (w * K_STEPS + k) * TILE
        ex = jnp.minimum(e0, N - TILE)
        r0 = (w * K_STEPS + k) * ROWS
        st = k % NSET
        pltpu.make_async_copy(x_hbm.at[pl.ds(ex, TILE)], xbufs[st], semL).wait()
        pltpu.make_async_copy(b2d_hbm.at[pl.ds(r0, ROWS)], idxbufs[st], semL).wait()

    fire_loads(0)
    fire_loads(1)

    onev = jnp.full((L,), 1.0, F32)
    zerov = jnp.zeros((L,), F32)

    def fill_ones(i, _):
        ones[pl.ds(i * L, L)] = onev
        return 0

    lax.fori_loop(0, 128 // L, fill_ones, 0)

    def fill_z(i, _):
        zbuf[pl.ds(i * L, L)] = zerov
        return 0

    lax.fori_loop(0, SEG_SLICE // L, fill_z, 0)
    pltpu.sync_copy(zbuf, acc_c.at[pl.ds(base, SEG_SLICE)])
    pltpu.sync_copy(zbuf, acc_s.at[pl.ds(base, SEG_SLICE)])
    pltpu.sync_copy(zbuf, acc_q.at[pl.ds(base, SEG_SLICE)])
    plsc.subcore_barrier()

    def fire_scatters(k):
        st = k % NSET

        def fire(j, _):
            idxr = idxbufs[st].at[j]
            pltpu.async_copy(ones, acc_c.at[idxr], semS, add=True)
            pltpu.async_copy(xbufs[st].at[pl.ds(j * 128, 128)], acc_s.at[idxr],
                             semS, add=True)
            pltpu.async_copy(xsqs[st].at[pl.ds(j * 128, 128)], acc_q.at[idxr],
                             semS, add=True)
            return 0

        lax.fori_loop(0, ROWS, fire, 0)

    def drain_scatters(k):
        st = k % NSET

        def drain(j, _):
            idxr = idxbufs[st].at[j]
            pltpu.make_async_copy(ones, acc_c.at[idxr], semS).wait()
            pltpu.make_async_copy(xbufs[st].at[pl.ds(j * 128, 128)],
                                  acc_s.at[idxr], semS).wait()
            pltpu.make_async_copy(xsqs[st].at[pl.ds(j * 128, 128)],
                                  acc_q.at[idxr], semS).wait()
            return 0

        lax.fori_loop(0, ROWS, drain, 0)

    def square(k):
        st = k % NSET

        def sq(i, _):
            v = xbufs[st][pl.ds(i * L, L)]
            xsqs[st][pl.ds(i * L, L)] = v * v
            return 0

        lax.fori_loop(0, TILE // L, sq, 0)

    for k in range(K_STEPS):
        wait_loads(k)
        square(k)
        if k >= 2:
            drain_scatters(k - 2)
        if k + 2 < K_STEPS:
            fire_loads(k + 2)
        fire_scatters(k)
    drain_scatters(K_STEPS - 2)
    drain_scatters(K_STEPS - 1)
    plsc.subcore_barrier()

    pltpu.sync_copy(acc_c.at[pl.ds(base, SEG_SLICE)], zbuf)
    pltpu.sync_copy(zbuf, part_hbm.at[pl.ds(c * 3 * SEG_PAD + base, SEG_SLICE)])
    pltpu.sync_copy(acc_s.at[pl.ds(base, SEG_SLICE)], zbuf)
    pltpu.sync_copy(zbuf, part_hbm.at[pl.ds((c * 3 + 1) * SEG_PAD + base, SEG_SLICE)])
    pltpu.sync_copy(acc_q.at[pl.ds(base, SEG_SLICE)], zbuf)
    pltpu.sync_copy(zbuf, part_hbm.at[pl.ds((c * 3 + 2) * SEG_PAD + base, SEG_SLICE)])


@functools.partial(
    pl.kernel,
    out_type=jax.ShapeDtypeStruct((N,), F32),
    mesh=_mesh,
    scratch_types=(
        [pltpu.VMEM((TILE,), F32) for _ in range(2)]
        + [pltpu.VMEM((TILE,), I32) for _ in range(2)]
        + [pltpu.VMEM((TILE,), I32) for _ in range(2)]
        + [pltpu.VMEM((TILE,), F32) for _ in range(2)]
        + [pltpu.VMEM((SUB,), F32) for _ in range(12)]
        + [
            pltpu.VMEM((SEG_SLICE,), I32),
            pltpu.VMEM((L,), F32),
            pltpu.VMEM((L,), F32),
            pltpu.VMEM_SHARED((SEG_PAD,), I32),
            pltpu.SemaphoreType.DMA,
            pltpu.SemaphoreType.DMA,
            pltpu.SemaphoreType.DMA,
            pltpu.SemaphoreType.DMA,
        ]
    ),
)
def _norm(x_hbm, b_hbm, part_hbm, gain_hbm, bias_hbm, out_hbm, *refs):
    xbufs = refs[0:2]
    idxbufs = refs[2:4]
    pbufs = refs[4:6]
    obufs = refs[6:8]
    psets = [refs[8:14], refs[14:20]]
    (ab_v, gb, bb, ab_sp, semL, semG, semO, semP) = refs[20:]
    c = lax.axis_index("c")
    s = lax.axis_index("s")
    w = c * NS + s

    def fire_loads(k):
        e0 = (w * K_STEPS + k) * TILE
        ex = jnp.minimum(e0, N - TILE)
        st = k % 2
        pltpu.async_copy(x_hbm.at[pl.ds(ex, TILE)], xbufs[st], semL)
        pltpu.async_copy(b_hbm.at[pl.ds(e0, TILE)], idxbufs[st], semL)

    def wait_loads(k):
        e0 = (w * K_STEPS + k) * TILE
        ex = jnp.minimum(e0, N - TILE)
        st = k % 2
        pltpu.make_async_copy(x_hbm.at[pl.ds(ex, TILE)], xbufs[st], semL).wait()
        pltpu.make_async_copy(b_hbm.at[pl.ds(e0, TILE)], idxbufs[st], semL).wait()

    fire_loads(0)
    pltpu.sync_copy(gain_hbm, gb)
    pltpu.sync_copy(bias_hbm, bb)
    gain = gb[...]
    bias = bb[...]

    def fire_parts(sub):
        sb = s * SEG_SLICE + sub * SUB
        ps = psets[sub % 2]
        for q in range(3):
            pltpu.async_copy(part_hbm.at[pl.ds(q * SEG_PAD + sb, SUB)], ps[2 * q], semP)
            pltpu.async_copy(part_hbm.at[pl.ds((3 + q) * SEG_PAD + sb, SUB)], ps[2 * q + 1], semP)

    def wait_parts(sub):
        sb = s * SEG_SLICE + sub * SUB
        ps = psets[sub % 2]
        for q in range(3):
            pltpu.make_async_copy(part_hbm.at[pl.ds(q * SEG_PAD + sb, SUB)], ps[2 * q], semP).wait()
            pltpu.make_async_copy(part_hbm.at[pl.ds((3 + q) * SEG_PAD + sb, SUB)], ps[2 * q + 1], semP).wait()

    fire_parts(0)
    for sub in range(SEG_SLICE // SUB):
        if sub + 1 < SEG_SLICE // SUB:
            fire_parts(sub + 1)
        wait_parts(sub)
        p0, p1, p2, p3, p4, p5 = psets[sub % 2]

        def seg(i, _):
            d = pl.ds(i * L, L)
            cnt = p0[d] + p1[d]
            cl = jnp.maximum(cnt, jnp.float32(1.0))
            sm = p2[d] + p3[d]
            q = p4[d] + p5[d]
            m = sm / cl
            var = jnp.maximum(q / cl - m * m, jnp.float32(0.0))
            vs = jnp.maximum(var, jnp.float32(1e-30))
            std = vs * _rsqrt(vs)
            a = gain / (std + jnp.float32(EPS))
            b = bias - m * a
            half = jnp.full((L,), 0x8000, I32)
            himask = jnp.full((L,), -65536, I32)
            s16 = jnp.full((L,), 16, I32)
            ar = lax.bitcast_convert_type(a, I32) + half
            br = lax.bitcast_convert_type(b, I32) + half
            packed = (ar & himask) | lax.shift_right_logical(br, s16)
            ab_v[pl.ds(sub * SUB + i * L, L)] = packed
            return 0

        lax.fori_loop(0, SUB // L, seg, 0)

    sl = pl.ds(s * SEG_SLICE, SEG_SLICE)
    pltpu.sync_copy(ab_v, ab_sp.at[sl])
    plsc.subcore_barrier()

    himask = jnp.full((L,), -65536, I32)
    s16 = jnp.full((L,), 16, I32)

    def fire_gathers(k, lo, hi):
        st = k % 2

        def fire(j, _):
            d = pl.ds(j * 128, 128)
            pltpu.async_copy(ab_sp.at[idxbufs[st].at[d]], pbufs[st].at[d], semG)
            return 0

        lax.fori_loop(lo, hi, fire, 0)

    def drain_gathers(k, lo, hi):
        st = k % 2

        def drain(j, _):
            d = pl.ds(j * 128, 128)
            pltpu.make_async_copy(ab_sp.at[idxbufs[st].at[d]], pbufs[st].at[d],
                                  semG).wait()
            return 0

        lax.fori_loop(lo, hi, drain, 0)

    def compute(k, lo, hi):
        st = k % 2

        def inner(i, _):
            d = pl.ds(i * L, L)
            p = pbufs[st][d]
            a = lax.bitcast_convert_type(p & himask, F32)
            b = lax.bitcast_convert_type(lax.shift_left(p, s16), F32)
            obufs[st][d] = xbufs[st][d] * a + b
            return 0

        lax.fori_loop(lo, hi, inner, 0)

    STRAD = 512

    def fire_store(k):
        st = k % 2
        e0 = (w * K_STEPS + k) * TILE

        @pl.when(e0 + TILE <= N)
        def _full():
            pltpu.async_copy(obufs[st], out_hbm.at[pl.ds(e0, TILE)], semO)

        @pl.when(jnp.logical_and(e0 < N, e0 + TILE > N))
        def _part():
            pltpu.async_copy(obufs[st].at[pl.ds(0, STRAD)],
                             out_hbm.at[pl.ds(e0, STRAD)], semO)

    def wait_store(k):
        st = k % 2
        e0 = (w * K_STEPS + k) * TILE

        @pl.when(e0 + TILE <= N)
        def _full():
            pltpu.make_async_copy(obufs[st], out_hbm.at[pl.ds(e0, TILE)],
                                  semO).wait()

        @pl.when(jnp.logical_and(e0 < N, e0 + TILE > N))
        def _part():
            pltpu.make_async_copy(obufs[st].at[pl.ds(0, STRAD)],
                                  out_hbm.at[pl.ds(e0, STRAD)], semO).wait()

    HALF = ROWS // 2
    for k in range(K_STEPS):
        wait_loads(k)
        fire_gathers(k, 0, HALF)
        if k + 1 < K_STEPS:
            fire_loads(k + 1)
        drain_gathers(k, 0, HALF)
        fire_gathers(k, HALF, ROWS)
        compute(k, 0, TILE // (2 * L))
        drain_gathers(k, HALF, ROWS)
        if k >= 2:
            wait_store(k - 2)
        compute(k, TILE // (2 * L), TILE // L)
        fire_store(k)
    wait_store(K_STEPS - 2)
    wait_store(K_STEPS - 1)


def kernel(inputs, batch, gain, bias):
    x = inputs.astype(F32)
    b = batch.astype(I32)
    pad = N_PAD - N
    xp = jnp.concatenate([x, jnp.zeros((pad,), F32)])
    dummy = NUM_SEG + (jnp.arange(pad, dtype=I32) % (SEG_PAD - NUM_SEG))
    bp = jnp.concatenate([b, dummy])
    b2d = bp.reshape(N_PAD // 128, 128)
    g16 = jnp.broadcast_to(gain.astype(F32), (L,))
    b16 = jnp.broadcast_to(bias.astype(F32), (L,))
    part = _stats(xp, b2d)
    outp = _norm(xp, bp, part, g16, b16)
    return outp.reshape(N, 1)
